# baseline (speedup 1.0000x reference)
import jax
import jax.numpy as jnp
from jax.experimental import pallas as pl


def _mi_tc_body(lt_ref, mk_ref, out_ref):
    a0 = lt_ref[0]
    a1 = lt_ref[1]
    a2 = lt_ref[2]
    mf = jnp.where(mk_ref[...] != 0, jnp.float32(1.0), jnp.float32(0.0))
    mx = jnp.maximum(a0, jnp.maximum(a1, a2))
    e0 = jnp.exp(a0 - mx)
    e1 = jnp.exp(a1 - mx)
    e2 = jnp.exp(a2 - mx)
    sinv = jnp.float32(1.0) / (e0 + e1 + e2)
    p0 = e0 * sinv
    p12 = (e1 + e2) * sinv
    h = -(p0 * jnp.log(p0) + p12 * jnp.log(p12))
    count = jnp.sum(mf)
    cinv = jnp.float32(1.0) / count
    condi = jnp.sum(h * mf) * cinv
    y0 = jnp.sum(p0 * mf) * cinv
    y1 = jnp.float32(1.0) - y0
    ye = -(y0 * jnp.log(y0) + y1 * jnp.log(y1))
    first = jnp.where(ye < jnp.float32(0.5), condi - ye, condi)
    out_ref[...] = jnp.concatenate(
        [jnp.broadcast_to(first, (1, 1)), jnp.broadcast_to(ye, (1, 1))],
        axis=1)


def kernel(logits, masks):
    lt = logits.transpose(2, 0, 1)
    mk = masks.astype(jnp.int32)
    out = pl.pallas_call(
        _mi_tc_body,
        out_shape=jax.ShapeDtypeStruct((1, 2), jnp.float32),
    )(lt, mk)
    return (out[0, 0], out[0, 1])


# speedup vs baseline: 1.4641x; 1.4641x over previous
import jax
import jax.numpy as jnp
from jax.experimental import pallas as pl


def _mi_tc_body(lt_ref, mk_ref, out_first, out_ye):
    a0 = lt_ref[0]
    a1 = lt_ref[1]
    a2 = lt_ref[2]
    mf = jnp.where(mk_ref[...] != 0, jnp.float32(1.0), jnp.float32(0.0))
    mx = jnp.maximum(a0, jnp.maximum(a1, a2))
    e0 = jnp.exp(a0 - mx)
    e1 = jnp.exp(a1 - mx)
    e2 = jnp.exp(a2 - mx)
    sinv = jnp.float32(1.0) / (e0 + e1 + e2)
    p0 = e0 * sinv
    p12 = (e1 + e2) * sinv
    h = -(p0 * jnp.log(p0) + p12 * jnp.log(p12))
    count = jnp.sum(mf)
    cinv = jnp.float32(1.0) / count
    condi = jnp.sum(h * mf) * cinv
    y0 = jnp.sum(p0 * mf) * cinv
    y1 = jnp.float32(1.0) - y0
    ye = -(y0 * jnp.log(y0) + y1 * jnp.log(y1))
    first = jnp.where(ye < jnp.float32(0.5), condi - ye, condi)
    out_first[...] = jnp.broadcast_to(first, (1, 1))
    out_ye[...] = jnp.broadcast_to(ye, (1, 1))


def kernel(logits, masks):
    lt = logits.transpose(2, 0, 1)
    mk = masks.astype(jnp.int32)
    first, ye = pl.pallas_call(
        _mi_tc_body,
        out_shape=(jax.ShapeDtypeStruct((1, 1), jnp.float32),
                   jax.ShapeDtypeStruct((1, 1), jnp.float32)),
    )(lt, mk)
    return (first[0, 0], ye[0, 0])
